# baseline (speedup 1.0000x reference)
import functools

import jax
import jax.numpy as jnp
from jax import lax
from jax.experimental import pallas as pl
from jax.experimental.pallas import tpu as pltpu
from jax.experimental.pallas import tpu_sc as plsc

_NWORDS = 1000000
_H = 1 << 19
_EMB = 64
_NTAGS = 1000
_BATCH = 4096
_HIST = 200
_TOTAL = _BATCH * _HIST

_NC = 2
_NS = 16
_NW = _NC * _NS
_IDX_ROWS = _TOTAL // 128 // _NS
_IDX_BATCH = _IDX_ROWS // 2
_ZCHUNK = 4096
_SLICE = _H // _NS

_CH = 256
_CPT = 122
_EXTRA_ROW0 = _CPT * _NW * _CH
_EXTRA_ROW1 = _EXTRA_ROW0 + _CH
_TAIL_ROW0 = _EXTRA_ROW1 + _CH
_TAIL = _NWORDS - _TAIL_ROW0


def _sc_hist_body(words_hbm, out_hbm, idx_v, tix_v, tval_v, zbuf, counts_sh):
    cid = lax.axis_index("c")
    sid = lax.axis_index("s")
    base = cid * _H

    zeros16 = jnp.zeros((16,), jnp.float32)

    def fill_z(i, _):
        zbuf[pl.ds(i * 16, 16)] = zeros16
        return 0

    lax.fori_loop(0, _ZCHUNK // 16, fill_z, 0)

    def zero_counts(k, _):
        pltpu.sync_copy(
            zbuf, counts_sh.at[pl.ds(sid * _SLICE + k * _ZCHUNK, _ZCHUNK)]
        )
        return 0

    lax.fori_loop(0, _SLICE // _ZCHUNK, zero_counts, 0)
    plsc.subcore_barrier()

    def scatter(j, _):
        for g in range(8):
            iv = idx_v[j, pl.ds(g * 16, 16)] - base
            inr = (iv >= 0) & (iv < _H)
            tix_v[pl.ds(g * 16, 16)] = jnp.minimum(
                jnp.maximum(iv, 0), _H - 1
            )
            tval_v[pl.ds(g * 16, 16)] = jnp.where(inr, 1.0, 0.0)
        pltpu.sync_copy(tval_v, counts_sh.at[tix_v], add=True)
        return 0

    for b in range(2):
        pltpu.sync_copy(
            words_hbm.at[pl.ds(sid * _IDX_ROWS + b * _IDX_BATCH, _IDX_BATCH)],
            idx_v,
        )
        lax.fori_loop(0, _IDX_BATCH, scatter, 0)
    plsc.subcore_barrier()

    pltpu.sync_copy(
        counts_sh.at[pl.ds(sid * _SLICE, _SLICE)],
        out_hbm.at[cid, pl.ds(sid * _SLICE, _SLICE)],
    )


_sc_hist = functools.partial(
    pl.kernel,
    mesh=plsc.VectorSubcoreMesh(core_axis_name="c", subcore_axis_name="s"),
    out_type=jax.ShapeDtypeStruct((_NC, _H), jnp.float32),
    scratch_types=[
        pltpu.VMEM((_IDX_BATCH, 128), jnp.int32),
        pltpu.VMEM((128,), jnp.int32),
        pltpu.VMEM((128,), jnp.float32),
        pltpu.VMEM((_ZCHUNK,), jnp.float32),
        pltpu.VMEM_SHARED((_H,), jnp.float32),
    ],
)(_sc_hist_body)


def _sc_stream_body(counts_hbm, table_hbm, out_hbm,
                    t_a, t_b, c_a, c_b, accbuf, sem_a, sem_b):
    cid = lax.axis_index("c")
    sid = lax.axis_index("s")
    wid = sid * _NC + cid

    def crow(r0):
        if isinstance(r0, int):
            h = 1 if r0 >= _H else 0
        else:
            h = (r0 >= _H).astype(jnp.int32)
        return h, r0 - h * _H

    def issue(j, t_v, c_v, sem):
        r0 = (j * _NW + wid) * _CH
        h, col = crow(r0)
        pltpu.async_copy(table_hbm.at[pl.ds(r0, _CH)], t_v, sem)
        pltpu.async_copy(counts_hbm.at[h, pl.ds(col, _CH)], c_v, sem)

    def waitset(j, t_v, c_v, sem):
        r0 = (j * _NW + wid) * _CH
        h, col = crow(r0)
        pltpu.make_async_copy(table_hbm.at[pl.ds(r0, _CH)], t_v, sem).wait()
        pltpu.make_async_copy(
            counts_hbm.at[h, pl.ds(col, _CH)], c_v, sem).wait()

    zeros_f = jnp.zeros((16,), jnp.float32)

    def proc(n_rows, t_v, c_v, acc):
        def group(g, acc):
            cvec = c_v[pl.ds(g * 16, 16)]
            for l in range(16):
                a0, a1, a2, a3 = acc
                cs = lax.gather(
                    cvec, jnp.full((16, 1), l, jnp.int32),
                    lax.GatherDimensionNumbers(
                        offset_dims=(), collapsed_slice_dims=(0,),
                        start_index_map=(0,)),
                    (1,),
                    mode=lax.GatherScatterMode.PROMISE_IN_BOUNDS,
                )
                i = g * 16 + l
                acc = (
                    a0 + t_v[i, pl.ds(0, 16)] * cs,
                    a1 + t_v[i, pl.ds(16, 16)] * cs,
                    a2 + t_v[i, pl.ds(32, 16)] * cs,
                    a3 + t_v[i, pl.ds(48, 16)] * cs,
                )
            return acc

        return lax.fori_loop(0, n_rows // 16, group, acc)

    acc = (zeros_f, zeros_f, zeros_f, zeros_f)
    issue(0, t_a, c_a, sem_a)

    def body(k, acc):
        issue(2 * k + 1, t_b, c_b, sem_b)
        waitset(2 * k, t_a, c_a, sem_a)
        acc = proc(_CH, t_a, c_a, acc)

        @pl.when(k < _CPT // 2 - 1)
        def _():
            issue(2 * k + 2, t_a, c_a, sem_a)

        waitset(2 * k + 1, t_b, c_b, sem_b)
        acc = proc(_CH, t_b, c_b, acc)
        return acc

    acc = lax.fori_loop(0, _CPT // 2, body, acc)

    a0, a1, a2, a3 = acc
    accbuf[...] = (a0 + a1) + (a2 + a3)

    for xw, xr0 in ((0, _EXTRA_ROW0), (1, _EXTRA_ROW1)):
        @pl.when(wid == xw)
        def _(xr0=xr0):
            h, col = crow(xr0)
            pltpu.async_copy(table_hbm.at[pl.ds(xr0, _CH)], t_a, sem_a)
            pltpu.async_copy(counts_hbm.at[h, pl.ds(col, _CH)], c_a, sem_a)
            pltpu.make_async_copy(
                table_hbm.at[pl.ds(xr0, _CH)], t_a, sem_a).wait()
            pltpu.make_async_copy(
                counts_hbm.at[h, pl.ds(col, _CH)], c_a, sem_a).wait()
            b0, b1, b2, b3 = proc(_CH, t_a, c_a,
                                  (zeros_f, zeros_f, zeros_f, zeros_f))
            accbuf[...] = accbuf[...] + (b0 + b1) + (b2 + b3)

    @pl.when(wid == 2)
    def _():
        h, col = crow(_TAIL_ROW0)
        pltpu.async_copy(table_hbm.at[pl.ds(_TAIL_ROW0, _TAIL)],
                         t_a.at[pl.ds(0, _TAIL)], sem_a)
        pltpu.async_copy(counts_hbm.at[h, pl.ds(col, _TAIL)],
                         c_a.at[pl.ds(0, _TAIL)], sem_a)
        pltpu.make_async_copy(
            table_hbm.at[pl.ds(_TAIL_ROW0, _TAIL)],
            t_a.at[pl.ds(0, _TAIL)], sem_a).wait()
        pltpu.make_async_copy(
            counts_hbm.at[h, pl.ds(col, _TAIL)],
            c_a.at[pl.ds(0, _TAIL)], sem_a).wait()
        b0, b1, b2, b3 = proc(_TAIL, t_a, c_a,
                              (zeros_f, zeros_f, zeros_f, zeros_f))
        accbuf[...] = accbuf[...] + (b0 + b1) + (b2 + b3)

    pltpu.sync_copy(accbuf, out_hbm.at[wid])


_sc_stream = functools.partial(
    pl.kernel,
    mesh=plsc.VectorSubcoreMesh(core_axis_name="c", subcore_axis_name="s"),
    out_type=jax.ShapeDtypeStruct((_NW, 16), jnp.float32),
    scratch_types=[
        pltpu.VMEM((_CH, _EMB), jnp.float32),
        pltpu.VMEM((_CH, _EMB), jnp.float32),
        pltpu.VMEM((_CH,), jnp.float32),
        pltpu.VMEM((_CH,), jnp.float32),
        pltpu.VMEM((16,), jnp.float32),
        pltpu.SemaphoreType.DMA,
        pltpu.SemaphoreType.DMA,
    ],
)(_sc_stream_body)


def _mlp_body(p_ref, w0_ref, b0_ref, w1_ref, b1_ref, wout_ref, bout_ref,
              o_ref):
    s = jnp.sum(p_ref[...])
    h = jnp.tanh(s * w0_ref[...] + b0_ref[...])
    h = jnp.tanh(
        lax.dot_general(
            h, w1_ref[...], (((1,), (0,)), ((), ())),
            preferred_element_type=jnp.float32,
            precision=lax.Precision.HIGHEST,
        )
        + b1_ref[...]
    )
    o_ref[...] = (
        lax.dot_general(
            h, wout_ref[...], (((1,), (0,)), ((), ())),
            preferred_element_type=jnp.float32,
            precision=lax.Precision.HIGHEST,
        )
        + bout_ref[...]
    )


_mlp = pl.pallas_call(
    _mlp_body,
    out_shape=jax.ShapeDtypeStruct((1, _NTAGS), jnp.float32),
)


def kernel(words, emb_table, W0, b0, W1, b1, Wout, bout):
    words2 = words.astype(jnp.int32).reshape(_TOTAL // 128, 128)
    counts = _sc_hist(words2)
    partials = _sc_stream(counts, emb_table)
    return _mlp(
        partials,
        W0,
        b0.reshape(1, _EMB),
        W1,
        b1.reshape(1, _EMB),
        Wout,
        bout.reshape(1, _NTAGS),
    )


# speedup vs baseline: 1.0856x; 1.0856x over previous
import functools

import jax
import jax.numpy as jnp
from jax import lax
from jax.experimental import pallas as pl
from jax.experimental.pallas import tpu as pltpu
from jax.experimental.pallas import tpu_sc as plsc

_NWORDS = 1000000
_H = 1 << 19
_EMB = 64
_NTAGS = 1000
_BATCH = 4096
_HIST = 200
_TOTAL = _BATCH * _HIST

_NC = 2
_NS = 16
_NW = _NC * _NS
_IDX_ROWS = _TOTAL // 128 // _NS
_NSET = 4
_ZCHUNK = 4096
_SLICE = _H // _NS

_CH = 256
_CPT = _H // _CH // _NW

_R = 8192
_TC_ROWS = _NWORDS - _H
_TC_NB = (_TC_ROWS + _R - 1) // _R
_TC_VALID = _TC_ROWS - (_TC_NB - 1) * _R


def _sc_hist_body(words_hbm, out_hbm, idx_v, tix_v, tval_v, zbuf,
                  counts_sh, sem):
    cid = lax.axis_index("c")
    sid = lax.axis_index("s")
    base = cid * _H

    zeros16 = jnp.zeros((16,), jnp.float32)

    def fill_z(i, _):
        zbuf[pl.ds(i * 16, 16)] = zeros16
        return 0

    lax.fori_loop(0, _ZCHUNK // 16, fill_z, 0)

    def zero_counts(k, _):
        pltpu.sync_copy(
            zbuf, counts_sh.at[pl.ds(sid * _SLICE + k * _ZCHUNK, _ZCHUNK)]
        )
        return 0

    lax.fori_loop(0, _SLICE // _ZCHUNK, zero_counts, 0)
    plsc.subcore_barrier()

    def transform(j, s):
        for g in range(8):
            iv = idx_v[j, pl.ds(g * 16, 16)] - base
            inr = (iv >= 0) & (iv < _H)
            tix_v[s, pl.ds(g * 16, 16)] = jnp.minimum(
                jnp.maximum(iv, 0), _H - 1
            )
            tval_v[s, pl.ds(g * 16, 16)] = jnp.where(inr, 1.0, 0.0)

    def scatter_group(j4, _):
        for s in range(_NSET):
            transform(j4 * _NSET + s, s)
            pltpu.async_copy(
                tval_v.at[s], counts_sh.at[tix_v.at[s]], sem, add=True
            )
        for s in range(_NSET):
            pltpu.make_async_copy(
                tval_v.at[s], counts_sh.at[tix_v.at[s]], sem
            ).wait()
        return 0

    for b in range(2):
        pltpu.sync_copy(
            words_hbm.at[pl.ds(sid * _IDX_ROWS + b * (_IDX_ROWS // 2),
                               _IDX_ROWS // 2)],
            idx_v,
        )
        lax.fori_loop(0, _IDX_ROWS // 2 // _NSET, scatter_group, 0)
    plsc.subcore_barrier()

    pltpu.sync_copy(
        counts_sh.at[pl.ds(sid * _SLICE, _SLICE)],
        out_hbm.at[cid, pl.ds(sid * _SLICE, _SLICE)],
    )


_sc_hist = functools.partial(
    pl.kernel,
    mesh=plsc.VectorSubcoreMesh(core_axis_name="c", subcore_axis_name="s"),
    out_type=jax.ShapeDtypeStruct((_NC, _H), jnp.float32),
    scratch_types=[
        pltpu.VMEM((_IDX_ROWS // 2, 128), jnp.int32),
        pltpu.VMEM((_NSET, 128), jnp.int32),
        pltpu.VMEM((_NSET, 128), jnp.float32),
        pltpu.VMEM((_ZCHUNK,), jnp.float32),
        pltpu.VMEM_SHARED((_H,), jnp.float32),
        pltpu.SemaphoreType.DMA,
    ],
)(_sc_hist_body)


def _sc_stream_body(counts_hbm, table_hbm, out_hbm,
                    t_a, t_b, c_a, c_b, accbuf, sem_a, sem_b):
    cid = lax.axis_index("c")
    sid = lax.axis_index("s")
    wid = sid * _NC + cid

    def issue(j, t_v, c_v, sem):
        r0 = (j * _NW + wid) * _CH
        pltpu.async_copy(table_hbm.at[pl.ds(r0, _CH)], t_v, sem)
        pltpu.async_copy(counts_hbm.at[0, pl.ds(r0, _CH)], c_v, sem)

    def waitset(j, t_v, c_v, sem):
        r0 = (j * _NW + wid) * _CH
        pltpu.make_async_copy(table_hbm.at[pl.ds(r0, _CH)], t_v, sem).wait()
        pltpu.make_async_copy(
            counts_hbm.at[0, pl.ds(r0, _CH)], c_v, sem).wait()

    zeros_f = jnp.zeros((16,), jnp.float32)

    def proc(t_v, c_v, acc):
        def group(g, acc):
            cvec = c_v[pl.ds(g * 16, 16)]
            for l in range(16):
                a0, a1, a2, a3 = acc
                cs = lax.gather(
                    cvec, jnp.full((16, 1), l, jnp.int32),
                    lax.GatherDimensionNumbers(
                        offset_dims=(), collapsed_slice_dims=(0,),
                        start_index_map=(0,)),
                    (1,),
                    mode=lax.GatherScatterMode.PROMISE_IN_BOUNDS,
                )
                i = g * 16 + l
                acc = (
                    a0 + t_v[i, pl.ds(0, 16)] * cs,
                    a1 + t_v[i, pl.ds(16, 16)] * cs,
                    a2 + t_v[i, pl.ds(32, 16)] * cs,
                    a3 + t_v[i, pl.ds(48, 16)] * cs,
                )
            return acc

        return lax.fori_loop(0, _CH // 16, group, acc)

    acc = (zeros_f, zeros_f, zeros_f, zeros_f)
    issue(0, t_a, c_a, sem_a)

    def body(k, acc):
        issue(2 * k + 1, t_b, c_b, sem_b)
        waitset(2 * k, t_a, c_a, sem_a)
        acc = proc(t_a, c_a, acc)

        @pl.when(k < _CPT // 2 - 1)
        def _():
            issue(2 * k + 2, t_a, c_a, sem_a)

        waitset(2 * k + 1, t_b, c_b, sem_b)
        acc = proc(t_b, c_b, acc)
        return acc

    acc = lax.fori_loop(0, _CPT // 2, body, acc)

    a0, a1, a2, a3 = acc
    accbuf[...] = (a0 + a1) + (a2 + a3)
    pltpu.sync_copy(accbuf, out_hbm.at[wid])


_sc_stream = functools.partial(
    pl.kernel,
    mesh=plsc.VectorSubcoreMesh(core_axis_name="c", subcore_axis_name="s"),
    out_type=jax.ShapeDtypeStruct((_NW, 16), jnp.float32),
    scratch_types=[
        pltpu.VMEM((_CH, _EMB), jnp.float32),
        pltpu.VMEM((_CH, _EMB), jnp.float32),
        pltpu.VMEM((_CH,), jnp.float32),
        pltpu.VMEM((_CH,), jnp.float32),
        pltpu.VMEM((16,), jnp.float32),
        pltpu.SemaphoreType.DMA,
        pltpu.SemaphoreType.DMA,
    ],
)(_sc_stream_body)


def _tc_body(c_ref, t_ref, o_ref, acc):
    g = pl.program_id(0)
    c_bf = c_ref[1:2, :].astype(jnp.bfloat16)

    def _dot(a, b):
        return lax.dot_general(
            a, b, (((1,), (0,)), ((), ())),
            preferred_element_type=jnp.float32,
        )

    def contrib(t):
        t_hi = t.astype(jnp.bfloat16)
        t_lo = (t - t_hi.astype(jnp.float32)).astype(jnp.bfloat16)
        return _dot(c_bf, t_hi) + _dot(c_bf, t_lo)

    @pl.when(g == 0)
    def _():
        acc[...] = jnp.zeros((1, _EMB), jnp.float32)

    @pl.when(g < _TC_NB - 1)
    def _():
        acc[...] += contrib(t_ref[...])

    @pl.when(g == _TC_NB - 1)
    def _():
        rows = lax.broadcasted_iota(jnp.int32, (_R, _EMB), 0)
        t = jnp.where(rows < _TC_VALID, t_ref[...], 0.0)
        acc[...] += contrib(t)
        o_ref[...] = acc[...]


_tc_stream = pl.pallas_call(
    _tc_body,
    grid=(_TC_NB,),
    in_specs=[
        pl.BlockSpec((_NC, _R), lambda g: (0, g)),
        pl.BlockSpec((_R, _EMB), lambda g: (_H // _R + g, 0)),
    ],
    out_specs=pl.BlockSpec((1, _EMB), lambda g: (0, 0)),
    out_shape=jax.ShapeDtypeStruct((1, _EMB), jnp.float32),
    scratch_shapes=[pltpu.VMEM((1, _EMB), jnp.float32)],
)


def _mlp_body(p_ref, cs_ref, w0_ref, b0_ref, w1_ref, b1_ref, wout_ref,
              bout_ref, o_ref):
    s = jnp.sum(p_ref[...]) + jnp.sum(cs_ref[...])
    h = jnp.tanh(s * w0_ref[...] + b0_ref[...])
    h = jnp.tanh(
        lax.dot_general(
            h, w1_ref[...], (((1,), (0,)), ((), ())),
            preferred_element_type=jnp.float32,
            precision=lax.Precision.HIGHEST,
        )
        + b1_ref[...]
    )
    o_ref[...] = (
        lax.dot_general(
            h, wout_ref[...], (((1,), (0,)), ((), ())),
            preferred_element_type=jnp.float32,
            precision=lax.Precision.HIGHEST,
        )
        + bout_ref[...]
    )


_mlp = pl.pallas_call(
    _mlp_body,
    out_shape=jax.ShapeDtypeStruct((1, _NTAGS), jnp.float32),
)


def kernel(words, emb_table, W0, b0, W1, b1, Wout, bout):
    words2 = words.astype(jnp.int32).reshape(_TOTAL // 128, 128)
    counts = _sc_hist(words2)
    partials = _sc_stream(counts, emb_table)
    colsum = _tc_stream(counts, emb_table)
    return _mlp(
        partials,
        colsum,
        W0,
        b0.reshape(1, _EMB),
        W1,
        b1.reshape(1, _EMB),
        Wout,
        bout.reshape(1, _NTAGS),
    )


# speedup vs baseline: 1.4400x; 1.3265x over previous
import functools

import jax
import jax.numpy as jnp
from jax import lax
from jax.experimental import pallas as pl
from jax.experimental.pallas import tpu as pltpu
from jax.experimental.pallas import tpu_sc as plsc

_NWORDS = 1000000
_NPAD = 1 << 20
_EMB = 64
_NTAGS = 1000
_BATCH = 4096
_HIST = 200
_TOTAL = _BATCH * _HIST

_NC = 2
_NS = 16
_NW = _NC * _NS
_PER_TILE = _TOTAL // _NW
_IDX_ROWS = _PER_TILE // 128
_NSET = 4
_ZCHUNK = 4096
_SLICE = _NPAD // _NS

_R = 16384
_NB = (_NWORDS + _R - 1) // _R


def _sc_hist_body(words_hbm, out_hbm, idx_v, ones_v, zbuf, counts_sh, sem):
    cid = lax.axis_index("c")
    sid = lax.axis_index("s")
    wid = sid * _NC + cid

    zeros16 = jnp.zeros((16,), jnp.float32)
    ones16 = jnp.full((16,), 1.0, jnp.float32)

    def fill_z(i, _):
        zbuf[pl.ds(i * 16, 16)] = zeros16
        return 0

    lax.fori_loop(0, _ZCHUNK // 16, fill_z, 0)

    def fill_o(i, _):
        ones_v[pl.ds(i * 16, 16)] = ones16
        return 0

    lax.fori_loop(0, 8, fill_o, 0)

    def zero_counts(k, _):
        pltpu.sync_copy(
            zbuf, counts_sh.at[pl.ds(sid * _SLICE + k * _ZCHUNK, _ZCHUNK)]
        )
        return 0

    lax.fori_loop(0, _SLICE // _ZCHUNK, zero_counts, 0)
    plsc.subcore_barrier()

    pltpu.sync_copy(words_hbm.at[pl.ds(wid * _IDX_ROWS, _IDX_ROWS)], idx_v)

    def scatter_group(j4, _):
        for s in range(_NSET):
            pltpu.async_copy(
                ones_v, counts_sh.at[idx_v.at[j4 * _NSET + s]], sem,
                add=True,
            )
        for s in range(_NSET):
            pltpu.make_async_copy(
                ones_v, counts_sh.at[idx_v.at[j4 * _NSET + s]], sem
            ).wait()
        return 0

    lax.fori_loop(0, _IDX_ROWS // _NSET, scatter_group, 0)
    plsc.subcore_barrier()

    pltpu.sync_copy(
        counts_sh.at[pl.ds(sid * _SLICE, _SLICE)],
        out_hbm.at[cid, pl.ds(sid * _SLICE, _SLICE)],
    )


_sc_hist = functools.partial(
    pl.kernel,
    mesh=plsc.VectorSubcoreMesh(core_axis_name="c", subcore_axis_name="s"),
    out_type=jax.ShapeDtypeStruct((_NC, _NPAD), jnp.float32),
    scratch_types=[
        pltpu.VMEM((_IDX_ROWS, 128), jnp.int32),
        pltpu.VMEM((128,), jnp.float32),
        pltpu.VMEM((_ZCHUNK,), jnp.float32),
        pltpu.VMEM_SHARED((_NPAD,), jnp.float32),
        pltpu.SemaphoreType.DMA,
    ],
)(_sc_hist_body)


def _tc_body(c_ref, t_ref, w0_ref, b0_ref, w1_ref, b1_ref, wout_ref,
             bout_ref, o_ref, acc):
    g = pl.program_id(0)
    c = c_ref[0:1, :] + c_ref[1:2, :]
    c_bf = c.astype(jnp.bfloat16)

    def _dot(a, b):
        return lax.dot_general(
            a, b, (((1,), (0,)), ((), ())),
            preferred_element_type=jnp.float32,
        )

    def contrib(t):
        t_hi = t.astype(jnp.bfloat16)
        t_lo = (t - t_hi.astype(jnp.float32)).astype(jnp.bfloat16)
        return _dot(c_bf, t_hi) + _dot(c_bf, t_lo)

    @pl.when(g == 0)
    def _():
        acc[...] = jnp.zeros((1, _EMB), jnp.float32)

    @pl.when(g < _NB - 1)
    def _():
        acc[...] += contrib(t_ref[...])

    @pl.when(g == _NB - 1)
    def _():
        valid = _NWORDS - (_NB - 1) * _R
        rows = lax.broadcasted_iota(jnp.int32, (_R, _EMB), 0)
        t = jnp.where(rows < valid, t_ref[...], 0.0)
        acc[...] += contrib(t)

        s = jnp.sum(acc[...])
        h = jnp.tanh(s * w0_ref[...] + b0_ref[...])
        h = jnp.tanh(
            lax.dot_general(
                h, w1_ref[...], (((1,), (0,)), ((), ())),
                preferred_element_type=jnp.float32,
                precision=lax.Precision.HIGHEST,
            )
            + b1_ref[...]
        )
        o_ref[...] = (
            lax.dot_general(
                h, wout_ref[...], (((1,), (0,)), ((), ())),
                preferred_element_type=jnp.float32,
                precision=lax.Precision.HIGHEST,
            )
            + bout_ref[...]
        )


_tc_reduce_mlp = pl.pallas_call(
    _tc_body,
    grid=(_NB,),
    in_specs=[
        pl.BlockSpec((_NC, _R), lambda g: (0, g)),
        pl.BlockSpec((_R, _EMB), lambda g: (g, 0)),
        pl.BlockSpec((1, _EMB), lambda g: (0, 0)),
        pl.BlockSpec((1, _EMB), lambda g: (0, 0)),
        pl.BlockSpec((_EMB, _EMB), lambda g: (0, 0)),
        pl.BlockSpec((1, _EMB), lambda g: (0, 0)),
        pl.BlockSpec((_EMB, _NTAGS), lambda g: (0, 0)),
        pl.BlockSpec((1, _NTAGS), lambda g: (0, 0)),
    ],
    out_specs=pl.BlockSpec((1, _NTAGS), lambda g: (0, 0)),
    out_shape=jax.ShapeDtypeStruct((1, _NTAGS), jnp.float32),
    scratch_shapes=[pltpu.VMEM((1, _EMB), jnp.float32)],
)


def kernel(words, emb_table, W0, b0, W1, b1, Wout, bout):
    words2 = words.astype(jnp.int32).reshape(_TOTAL // 128, 128)
    counts = _sc_hist(words2)
    return _tc_reduce_mlp(
        counts,
        emb_table,
        W0,
        b0.reshape(1, _EMB),
        W1,
        b1.reshape(1, _EMB),
        Wout,
        bout.reshape(1, _NTAGS),
    )
